# Initial kernel scaffold; baseline (speedup 1.0000x reference)
import jax
import jax.numpy as jnp
from jax.experimental import pallas as pl


def kernel(x, edge_index, edge_attr, batch, W0, b0, We1, be1, We2, be2, b_conv, Wih_g, bih_g, Whh_g, bhh_g, Wih_l, bih_l, Whh_l, bhh_l, W1, b1, W2, b2):
    raise NotImplementedError("write your pallas kernel here")


# speedup vs baseline: 2.4634x; 2.4634x over previous
import jax
import jax.numpy as jnp
from jax import lax
from jax.experimental import pallas as pl
from jax.experimental.pallas import tpu as pltpu
from jax.experimental.pallas import tpu_sc as plsc

DIM = 32
NB = 64
NC = 2
NS = 16
NW = NC * NS
CHUNK = 128


def _lin0_body(x_ref, w_ref, b_ref, o_ref):
    n = x_ref.shape[0]
    res = jnp.maximum(
        jnp.dot(x_ref[...], w_ref[...], preferred_element_type=jnp.float32)
        + b_ref[...], 0.0)
    o_ref[...] = jnp.concatenate(
        [res, jnp.zeros((n, 128 - DIM), jnp.float32)], axis=1)


def _msg_body(ea_ref, xj_ref, we1_ref, be1_ref, we2_ref, s_ref, r4_ref,
              bm_ref, o_ref):
    ea = ea_ref[...]
    x = xj_ref[:, 0:DIM]
    h = jnp.maximum(
        ea[:, 0:1] * we1_ref[0:1, :] + ea[:, 1:2] * we1_ref[1:2, :]
        + be1_ref[...], 0.0)
    g = jnp.dot(h, we2_ref[...], preferred_element_type=jnp.float32)
    x3 = jnp.dot(x, s_ref[...], preferred_element_type=jnp.float32)
    p = g * x3
    kk = p.shape[1]
    m = p[:, 0:128]
    for t in range(1, kk // 128):
        m = m + p[:, t * 128:(t + 1) * 128]
    msg = (jnp.dot(m, r4_ref[...], preferred_element_type=jnp.float32)
           + jnp.dot(x, bm_ref[...], preferred_element_type=jnp.float32))
    blk = msg.shape[0]
    o_ref[...] = jnp.concatenate(
        [msg, jnp.ones((blk, 1), jnp.float32),
         jnp.zeros((blk, 127 - DIM), jnp.float32)], axis=1)


def _gru_body(p0_ref, p1_ref, h_ref, bc_ref, wih_ref,
              bih_ref, whh_ref, bhh_ref, o_ref):
    n = o_ref.shape[0]
    d = DIM
    h = h_ref[:, 0:d]
    agg = p0_ref[0:n, 0:d] + p1_ref[0:n, 0:d]
    cnt = p0_ref[0:n, d:d + 1] + p1_ref[0:n, d:d + 1]
    deg = jnp.maximum(cnt, 1.0)
    m = jnp.maximum(agg / deg + bc_ref[...], 0.0)
    gi = jnp.dot(m, wih_ref[...], preferred_element_type=jnp.float32) + bih_ref[...]
    gh = jnp.dot(h, whh_ref[...], preferred_element_type=jnp.float32) + bhh_ref[...]
    r = jax.nn.sigmoid(gi[:, 0:d] + gh[:, 0:d])
    z = jax.nn.sigmoid(gi[:, d:2 * d] + gh[:, d:2 * d])
    nn_ = jnp.tanh(gi[:, 2 * d:3 * d] + r * gh[:, 2 * d:3 * d])
    hn = (1.0 - z) * nn_ + z * h
    o_ref[...] = jnp.concatenate(
        [hn, jnp.zeros((n, 128 - d), jnp.float32)], axis=1)


def _s2s_body(out_ref, b_ref, wih_ref, bih_ref, whh_ref, bhh_ref, w1_ref,
              b1_ref, w2_ref, b2_ref, o_ref, nb, psteps):
    d = DIM
    outv = out_ref[:, 0:d]
    bidx = b_ref[...]
    oh = (bidx == lax.broadcasted_iota(jnp.int32, (1, nb), 1)
          ).astype(jnp.float32)
    q_star = jnp.zeros((nb, 2 * d), jnp.float32)
    hl = jnp.zeros((nb, d), jnp.float32)
    cl = jnp.zeros((nb, d), jnp.float32)
    neg_inf = jnp.float32(-jnp.inf)
    for _ in range(psteps):
        gates = (jnp.dot(q_star, wih_ref[...], preferred_element_type=jnp.float32)
                 + bih_ref[...]
                 + jnp.dot(hl, whh_ref[...], preferred_element_type=jnp.float32)
                 + bhh_ref[...])
        i_g = jax.nn.sigmoid(gates[:, 0:d])
        f_g = jax.nn.sigmoid(gates[:, d:2 * d])
        g_g = jnp.tanh(gates[:, 2 * d:3 * d])
        o_g = jax.nn.sigmoid(gates[:, 3 * d:4 * d])
        cl = f_g * cl + i_g * g_g
        hl = o_g * jnp.tanh(cl)
        q = hl
        qn = jnp.dot(oh, q, preferred_element_type=jnp.float32)
        e = jnp.sum(outv * qn, axis=1, keepdims=True)
        emax = jnp.max(jnp.where(oh > 0.5, e, neg_inf), axis=0, keepdims=True)
        emax = jnp.where(emax == neg_inf, 0.0, emax)
        en = jnp.sum(oh * emax, axis=1, keepdims=True)
        a = jnp.exp(e - en)
        asum = jnp.sum(oh * a, axis=0, keepdims=True)
        an = jnp.sum(oh * asum, axis=1, keepdims=True)
        a = a / (an + 1e-16)
        r_ = lax.dot_general(oh, a * outv, (((0,), (0,)), ((), ())),
                             preferred_element_type=jnp.float32)
        q_star = jnp.concatenate([q, r_], axis=1)
    o1 = jnp.maximum(
        jnp.dot(q_star, w1_ref[...], preferred_element_type=jnp.float32)
        + b1_ref[...], 0.0)
    logits = jnp.dot(o1, w2_ref[...], preferred_element_type=jnp.float32) + b2_ref[...]
    lmax = jnp.max(logits, axis=1, keepdims=True)
    sh = logits - lmax
    lse = jnp.log(jnp.sum(jnp.exp(sh), axis=1, keepdims=True))
    o_ref[...] = sh - lse


def _run_lin0(x, w0, b0):
    n = x.shape[0]
    return pl.pallas_call(
        _lin0_body,
        out_shape=jax.ShapeDtypeStruct((n, 128), jnp.float32),
    )(x, w0, b0.reshape(1, -1))


def _run_msg(ea_p, xj, we1, be1, we2, s, r4, bm):
    ep = ea_p.shape[0]
    blk = 1024
    hid = we1.shape[1]
    kk = we2.shape[1]
    return pl.pallas_call(
        _msg_body,
        grid=(ep // blk,),
        in_specs=[
            pl.BlockSpec((blk, 2), lambda i: (i, 0)),
            pl.BlockSpec((blk, 128), lambda i: (i, 0)),
            pl.BlockSpec((2, hid), lambda i: (0, 0)),
            pl.BlockSpec((1, hid), lambda i: (0, 0)),
            pl.BlockSpec((hid, kk), lambda i: (0, 0)),
            pl.BlockSpec((DIM, kk), lambda i: (0, 0)),
            pl.BlockSpec((128, DIM), lambda i: (0, 0)),
            pl.BlockSpec((DIM, DIM), lambda i: (0, 0)),
        ],
        out_specs=pl.BlockSpec((blk, 128), lambda i: (i, 0)),
        out_shape=jax.ShapeDtypeStruct((ep, 128), jnp.float32),
    )(ea_p, xj, we1, be1.reshape(1, -1), we2, s, r4, bm)


def _run_gru(p0, p1, h, bc, wih, bih, whh, bhh):
    n = h.shape[0]
    return pl.pallas_call(
        _gru_body,
        out_shape=jax.ShapeDtypeStruct((n, 128), jnp.float32),
    )(p0, p1, h, bc.reshape(1, -1), wih, bih.reshape(1, -1),
      whh, bhh.reshape(1, -1))


def _run_s2s(out, batch2d, wih, bih, whh, bhh, w1, b1, w2, b2, psteps):
    nb = NB
    ncls = w2.shape[1]

    def body(*refs):
        _s2s_body(*refs, nb=nb, psteps=psteps)

    return pl.pallas_call(
        body,
        out_shape=jax.ShapeDtypeStruct((nb, ncls), jnp.float32),
    )(out, batch2d, wih, bih.reshape(1, -1), whh, bhh.reshape(1, -1),
      w1, b1.reshape(1, -1), w2, b2.reshape(1, -1))


def _run_gather(table, src_p):
    ep = src_p.shape[0]
    cpw = ep // (NW * CHUNK)
    epw = cpw * CHUNK

    def body(table_ref, src_ref, out_ref, idx_v, rows_v, sem):
        cid = lax.axis_index("c")
        sid = lax.axis_index("s")
        base = (sid * NC + cid) * epw

        def step(c, carry):
            off = base + c * CHUNK
            pltpu.sync_copy(src_ref.at[pl.ds(off, CHUNK)], idx_v)
            pltpu.async_copy(table_ref.at[idx_v], rows_v, sem).wait()
            pltpu.sync_copy(rows_v, out_ref.at[pl.ds(off, CHUNK)])
            return carry

        lax.fori_loop(0, cpw, step, 0)

    mesh = plsc.VectorSubcoreMesh(core_axis_name="c", subcore_axis_name="s")
    return pl.kernel(
        body,
        out_type=jax.ShapeDtypeStruct((ep, 128), jnp.float32),
        mesh=mesh,
        scratch_types=[
            pltpu.VMEM((CHUNK,), jnp.int32),
            pltpu.VMEM((CHUNK, 128), jnp.float32),
            pltpu.SemaphoreType.DMA,
        ],
    )(table, src_p)


def _run_scatter(msg, dst_p, z128, npad):
    ep = msg.shape[0]
    cpw = ep // (NW * CHUNK)
    epw = cpw * CHUNK
    rpt = npad // NS

    def body(msg_ref, dst_ref, z_ref, outa_ref, agg_sh, idx_v, rows_v, sem):
        cid = lax.axis_index("c")
        sid = lax.axis_index("s")
        r0 = sid * rpt
        pltpu.sync_copy(z_ref.at[pl.ds(r0, rpt)], agg_sh.at[pl.ds(r0, rpt)])
        plsc.subcore_barrier()
        base = (sid * NC + cid) * epw

        def step(c, carry):
            off = base + c * CHUNK
            pltpu.sync_copy(dst_ref.at[pl.ds(off, CHUNK)], idx_v)
            pltpu.sync_copy(msg_ref.at[pl.ds(off, CHUNK)], rows_v)
            pltpu.sync_copy(rows_v, agg_sh.at[idx_v], add=True)
            return carry

        lax.fori_loop(0, cpw, step, 0)
        plsc.subcore_barrier()
        pltpu.sync_copy(agg_sh.at[pl.ds(r0, rpt)],
                        outa_ref.at[cid, pl.ds(r0, rpt)])

    mesh = plsc.VectorSubcoreMesh(core_axis_name="c", subcore_axis_name="s")
    return pl.kernel(
        body,
        out_type=jax.ShapeDtypeStruct((NC, npad, 128), jnp.float32),
        mesh=mesh,
        scratch_types=[
            pltpu.VMEM_SHARED((npad, 128), jnp.float32),
            pltpu.VMEM((CHUNK,), jnp.int32),
            pltpu.VMEM((CHUNK, 128), jnp.float32),
            pltpu.SemaphoreType.DMA,
        ],
    )(msg, dst_p, z128)


def kernel(x, edge_index, edge_attr, batch, W0, b0, We1, be1, We2, be2,
           b_conv, Wih_g, bih_g, Whh_g, bhh_g, Wih_l, bih_l, Whh_l, bhh_l,
           W1, b1, W2, b2):
    n = x.shape[0]
    e = edge_index.shape[1]
    kk = We2.shape[1]
    mp_steps = 2
    psteps = 4

    gran = NW * CHUNK
    ep = -(-e // gran) * gran
    pad = ep - e
    src_p = jnp.concatenate([edge_index[0], jnp.zeros((pad,), jnp.int32)])
    dst_p = jnp.concatenate([edge_index[1], jnp.full((pad,), n, jnp.int32)])
    ea_p = jnp.concatenate([edge_attr, jnp.zeros((pad, 2), jnp.float32)])
    npad = -(-(n + 1) // (NS * 8)) * (NS * 8)

    s = (jnp.arange(kk, dtype=jnp.int32)[None, :] // DIM
         == jnp.arange(DIM, dtype=jnp.int32)[:, None]).astype(jnp.float32)
    r4 = (jnp.arange(128, dtype=jnp.int32)[:, None] % DIM
          == jnp.arange(DIM, dtype=jnp.int32)[None, :]).astype(jnp.float32)
    bm = be2.reshape(DIM, DIM)
    z128 = jnp.zeros((npad, 128), jnp.float32)

    h = _run_lin0(x, W0, b0)
    for _ in range(mp_steps):
        xj = _run_gather(h, src_p)
        msg = _run_msg(ea_p, xj, We1, be1, We2, s, r4, bm)
        agg2 = _run_scatter(msg, dst_p, z128, npad)
        h = _run_gru(agg2[0], agg2[1], h, b_conv,
                     Wih_g, bih_g, Whh_g, bhh_g)

    return _run_s2s(h, batch.reshape(-1, 1), Wih_l, bih_l, Whh_l, bhh_l,
                    W1, b1, W2, b2, psteps)


# speedup vs baseline: 2.8086x; 1.1401x over previous
import jax
import jax.numpy as jnp
from jax import lax
from jax.experimental import pallas as pl
from jax.experimental.pallas import tpu as pltpu
from jax.experimental.pallas import tpu_sc as plsc

DIM = 32
NB = 64
NC = 2
NS = 16
NW = NC * NS
CHUNK = 128
NBUF = 4


def _lin0_body(x_ref, w_ref, b_ref, o_ref):
    n = x_ref.shape[0]
    res = jnp.maximum(
        jnp.dot(x_ref[...], w_ref[...], preferred_element_type=jnp.float32)
        + b_ref[...], 0.0)
    o_ref[...] = jnp.concatenate(
        [res, jnp.zeros((n, 128 - DIM), jnp.float32)], axis=1)


def _msg_body(ea_ref, xj_ref, we1_ref, be1_ref, we2_ref, s_ref, r4_ref,
              bm_ref, o_ref):
    ea = ea_ref[...]
    x = xj_ref[:, 0:DIM]
    h = jnp.maximum(
        ea[:, 0:1] * we1_ref[0:1, :] + ea[:, 1:2] * we1_ref[1:2, :]
        + be1_ref[...], 0.0)
    g = jnp.dot(h, we2_ref[...], preferred_element_type=jnp.float32)
    x3 = jnp.dot(x, s_ref[...], preferred_element_type=jnp.float32)
    p = g * x3
    kk = p.shape[1]
    m = p[:, 0:128]
    for t in range(1, kk // 128):
        m = m + p[:, t * 128:(t + 1) * 128]
    msg = (jnp.dot(m, r4_ref[...], preferred_element_type=jnp.float32)
           + jnp.dot(x, bm_ref[...], preferred_element_type=jnp.float32))
    blk = msg.shape[0]
    o_ref[...] = jnp.concatenate(
        [msg, jnp.ones((blk, 1), jnp.float32),
         jnp.zeros((blk, 127 - DIM), jnp.float32)], axis=1)


def _gru_body(p0_ref, p1_ref, h_ref, bc_ref, wih_ref,
              bih_ref, whh_ref, bhh_ref, o_ref):
    n = o_ref.shape[0]
    d = DIM
    h = h_ref[:, 0:d]
    agg = p0_ref[0:n, 0:d] + p1_ref[0:n, 0:d]
    cnt = p0_ref[0:n, d:d + 1] + p1_ref[0:n, d:d + 1]
    deg = jnp.maximum(cnt, 1.0)
    m = jnp.maximum(agg / deg + bc_ref[...], 0.0)
    gi = jnp.dot(m, wih_ref[...], preferred_element_type=jnp.float32) + bih_ref[...]
    gh = jnp.dot(h, whh_ref[...], preferred_element_type=jnp.float32) + bhh_ref[...]
    r = jax.nn.sigmoid(gi[:, 0:d] + gh[:, 0:d])
    z = jax.nn.sigmoid(gi[:, d:2 * d] + gh[:, d:2 * d])
    nn_ = jnp.tanh(gi[:, 2 * d:3 * d] + r * gh[:, 2 * d:3 * d])
    hn = (1.0 - z) * nn_ + z * h
    o_ref[...] = jnp.concatenate(
        [hn, jnp.zeros((n, 128 - d), jnp.float32)], axis=1)


def _s2s_body(out_ref, b_ref, wih_ref, bih_ref, whh_ref, bhh_ref, w1_ref,
              b1_ref, w2_ref, b2_ref, o_ref, nb, psteps):
    d = DIM
    outv = out_ref[:, 0:d]
    bidx = b_ref[...]
    oh = (bidx == lax.broadcasted_iota(jnp.int32, (1, nb), 1)
          ).astype(jnp.float32)
    q_star = jnp.zeros((nb, 2 * d), jnp.float32)
    hl = jnp.zeros((nb, d), jnp.float32)
    cl = jnp.zeros((nb, d), jnp.float32)
    neg_inf = jnp.float32(-jnp.inf)
    for _ in range(psteps):
        gates = (jnp.dot(q_star, wih_ref[...], preferred_element_type=jnp.float32)
                 + bih_ref[...]
                 + jnp.dot(hl, whh_ref[...], preferred_element_type=jnp.float32)
                 + bhh_ref[...])
        i_g = jax.nn.sigmoid(gates[:, 0:d])
        f_g = jax.nn.sigmoid(gates[:, d:2 * d])
        g_g = jnp.tanh(gates[:, 2 * d:3 * d])
        o_g = jax.nn.sigmoid(gates[:, 3 * d:4 * d])
        cl = f_g * cl + i_g * g_g
        hl = o_g * jnp.tanh(cl)
        q = hl
        qn = jnp.dot(oh, q, preferred_element_type=jnp.float32)
        e = jnp.sum(outv * qn, axis=1, keepdims=True)
        emax = jnp.max(jnp.where(oh > 0.5, e, neg_inf), axis=0, keepdims=True)
        emax = jnp.where(emax == neg_inf, 0.0, emax)
        en = jnp.sum(oh * emax, axis=1, keepdims=True)
        a = jnp.exp(e - en)
        asum = jnp.sum(oh * a, axis=0, keepdims=True)
        an = jnp.sum(oh * asum, axis=1, keepdims=True)
        a = a / (an + 1e-16)
        r_ = lax.dot_general(oh, a * outv, (((0,), (0,)), ((), ())),
                             preferred_element_type=jnp.float32)
        q_star = jnp.concatenate([q, r_], axis=1)
    o1 = jnp.maximum(
        jnp.dot(q_star, w1_ref[...], preferred_element_type=jnp.float32)
        + b1_ref[...], 0.0)
    logits = jnp.dot(o1, w2_ref[...], preferred_element_type=jnp.float32) + b2_ref[...]
    lmax = jnp.max(logits, axis=1, keepdims=True)
    sh = logits - lmax
    lse = jnp.log(jnp.sum(jnp.exp(sh), axis=1, keepdims=True))
    o_ref[...] = sh - lse


def _run_lin0(x, w0, b0):
    n = x.shape[0]
    return pl.pallas_call(
        _lin0_body,
        out_shape=jax.ShapeDtypeStruct((n, 128), jnp.float32),
    )(x, w0, b0.reshape(1, -1))


def _run_msg(ea_p, xj, we1, be1, we2, s, r4, bm):
    ep = ea_p.shape[0]
    blk = 1024
    hid = we1.shape[1]
    kk = we2.shape[1]
    return pl.pallas_call(
        _msg_body,
        grid=(ep // blk,),
        in_specs=[
            pl.BlockSpec((blk, 2), lambda i: (i, 0)),
            pl.BlockSpec((blk, 128), lambda i: (i, 0)),
            pl.BlockSpec((2, hid), lambda i: (0, 0)),
            pl.BlockSpec((1, hid), lambda i: (0, 0)),
            pl.BlockSpec((hid, kk), lambda i: (0, 0)),
            pl.BlockSpec((DIM, kk), lambda i: (0, 0)),
            pl.BlockSpec((128, DIM), lambda i: (0, 0)),
            pl.BlockSpec((DIM, DIM), lambda i: (0, 0)),
        ],
        out_specs=pl.BlockSpec((blk, 128), lambda i: (i, 0)),
        out_shape=jax.ShapeDtypeStruct((ep, 128), jnp.float32),
    )(ea_p, xj, we1, be1.reshape(1, -1), we2, s, r4, bm)


def _run_gru(p0, p1, h, bc, wih, bih, whh, bhh):
    n = h.shape[0]
    return pl.pallas_call(
        _gru_body,
        out_shape=jax.ShapeDtypeStruct((n, 128), jnp.float32),
    )(p0, p1, h, bc.reshape(1, -1), wih, bih.reshape(1, -1),
      whh, bhh.reshape(1, -1))


def _run_s2s(out, batch2d, wih, bih, whh, bhh, w1, b1, w2, b2, psteps):
    nb = NB
    ncls = w2.shape[1]

    def body(*refs):
        _s2s_body(*refs, nb=nb, psteps=psteps)

    return pl.pallas_call(
        body,
        out_shape=jax.ShapeDtypeStruct((nb, ncls), jnp.float32),
    )(out, batch2d, wih, bih.reshape(1, -1), whh, bhh.reshape(1, -1),
      w1, b1.reshape(1, -1), w2, b2.reshape(1, -1))


def _run_gather(table, src3):
    _, cpw, _ = src3.shape
    epw = cpw * CHUNK
    ngrp = cpw // NBUF

    def body(table_ref, src_ref, out_ref, idx2, bufs, sem_g, sem_s):
        cid = lax.axis_index("c")
        sid = lax.axis_index("s")
        wid = sid * NC + cid
        base = wid * epw
        pltpu.sync_copy(src_ref.at[wid], idx2)

        def group(k, carry):
            g0 = k * NBUF
            ds = [pltpu.async_copy(table_ref.at[idx2.at[g0 + b]],
                                   bufs.at[b], sem_g)
                  for b in range(NBUF)]
            ss = []
            for b in range(NBUF):
                ds[b].wait()
                off = base + (g0 + b) * CHUNK
                ss.append(pltpu.async_copy(
                    bufs.at[b], out_ref.at[pl.ds(off, CHUNK)], sem_s))
            for s_ in ss:
                s_.wait()
            return carry

        lax.fori_loop(0, ngrp, group, 0)

    mesh = plsc.VectorSubcoreMesh(core_axis_name="c", subcore_axis_name="s")
    return pl.kernel(
        body,
        out_type=jax.ShapeDtypeStruct((NW * epw, 128), jnp.float32),
        mesh=mesh,
        scratch_types=[
            pltpu.VMEM((cpw, CHUNK), jnp.int32),
            pltpu.VMEM((NBUF, CHUNK, 128), jnp.float32),
            pltpu.SemaphoreType.DMA,
            pltpu.SemaphoreType.DMA,
        ],
    )(table, src3)


def _run_scatter(msg, dst3, z128, npad):
    ep = msg.shape[0]
    _, cpw, _ = dst3.shape
    epw = cpw * CHUNK
    nbuf = 2
    ngrp = cpw // nbuf
    rpt = npad // NS

    def body(msg_ref, dst_ref, z_ref, outa_ref, agg_sh, idx2, bufs,
             sem_m, sem_w):
        cid = lax.axis_index("c")
        sid = lax.axis_index("s")
        wid = sid * NC + cid
        r0 = sid * rpt
        pltpu.sync_copy(z_ref.at[pl.ds(r0, rpt)], agg_sh.at[pl.ds(r0, rpt)])
        pltpu.sync_copy(dst_ref.at[wid], idx2)
        plsc.subcore_barrier()
        base = wid * epw

        def group(k, carry):
            g0 = k * nbuf
            ds = [pltpu.async_copy(
                      msg_ref.at[pl.ds(base + (g0 + b) * CHUNK, CHUNK)],
                      bufs.at[b], sem_m)
                  for b in range(nbuf)]
            ws = []
            for b in range(nbuf):
                ds[b].wait()
                ws.append(pltpu.async_copy(
                    bufs.at[b], agg_sh.at[idx2.at[g0 + b]], sem_w, add=True))
            for w_ in ws:
                w_.wait()
            return carry

        lax.fori_loop(0, ngrp, group, 0)
        plsc.subcore_barrier()
        pltpu.sync_copy(agg_sh.at[pl.ds(r0, rpt)],
                        outa_ref.at[cid, pl.ds(r0, rpt)])

    mesh = plsc.VectorSubcoreMesh(core_axis_name="c", subcore_axis_name="s")
    return pl.kernel(
        body,
        out_type=jax.ShapeDtypeStruct((NC, npad, 128), jnp.float32),
        mesh=mesh,
        scratch_types=[
            pltpu.VMEM_SHARED((npad, 128), jnp.float32),
            pltpu.VMEM((cpw, CHUNK), jnp.int32),
            pltpu.VMEM((nbuf, CHUNK, 128), jnp.float32),
            pltpu.SemaphoreType.DMA,
            pltpu.SemaphoreType.DMA,
        ],
    )(msg, dst3, z128)


def kernel(x, edge_index, edge_attr, batch, W0, b0, We1, be1, We2, be2,
           b_conv, Wih_g, bih_g, Whh_g, bhh_g, Wih_l, bih_l, Whh_l, bhh_l,
           W1, b1, W2, b2):
    n = x.shape[0]
    e = edge_index.shape[1]
    kk = We2.shape[1]
    mp_steps = 2
    psteps = 4

    gran = NW * CHUNK * NBUF
    ep = -(-e // gran) * gran
    pad = ep - e
    cpw = ep // (NW * CHUNK)
    src3 = jnp.concatenate([edge_index[0], jnp.zeros((pad,), jnp.int32)]
                           ).reshape(NW, cpw, CHUNK)
    dst3 = jnp.concatenate([edge_index[1], jnp.full((pad,), n, jnp.int32)]
                           ).reshape(NW, cpw, CHUNK)
    ea_p = jnp.concatenate([edge_attr, jnp.zeros((pad, 2), jnp.float32)])
    npad = -(-(n + 1) // (NS * 8)) * (NS * 8)

    s = (jnp.arange(kk, dtype=jnp.int32)[None, :] // DIM
         == jnp.arange(DIM, dtype=jnp.int32)[:, None]).astype(jnp.float32)
    r4 = (jnp.arange(128, dtype=jnp.int32)[:, None] % DIM
          == jnp.arange(DIM, dtype=jnp.int32)[None, :]).astype(jnp.float32)
    bm = be2.reshape(DIM, DIM)
    z128 = jnp.zeros((npad, 128), jnp.float32)

    h = _run_lin0(x, W0, b0)
    for _ in range(mp_steps):
        xj = _run_gather(h, src3)
        msg = _run_msg(ea_p, xj, We1, be1, We2, s, r4, bm)
        agg2 = _run_scatter(msg, dst3, z128, npad)
        h = _run_gru(agg2[0], agg2[1], h, b_conv,
                     Wih_g, bih_g, Whh_g, bhh_g)

    return _run_s2s(h, batch.reshape(-1, 1), Wih_l, bih_l, Whh_l, bhh_l,
                    W1, b1, W2, b2, psteps)


# speedup vs baseline: 2.9025x; 1.0334x over previous
import jax
import jax.numpy as jnp
from jax import lax
from jax.experimental import pallas as pl
from jax.experimental.pallas import tpu as pltpu
from jax.experimental.pallas import tpu_sc as plsc

DIM = 32
NB = 64
NC = 2
NS = 16
NW = NC * NS
CHUNK = 128
NBUF = 4


def _lin0_body(x_ref, w_ref, b_ref, o_ref):
    n = x_ref.shape[0]
    res = jnp.maximum(
        jnp.dot(x_ref[...], w_ref[...], preferred_element_type=jnp.float32)
        + b_ref[...], 0.0)
    o_ref[...] = jnp.concatenate(
        [res, jnp.zeros((n, 128 - DIM), jnp.float32)], axis=1)


def _msg_body(ea_ref, xj_ref, we1_ref, be1_ref, we2_ref, s_ref, r4_ref,
              bm_ref, o_ref):
    ea = ea_ref[...]
    x = xj_ref[:, 0:DIM]
    h = jnp.maximum(
        ea[:, 0:1] * we1_ref[0:1, :] + ea[:, 1:2] * we1_ref[1:2, :]
        + be1_ref[...], 0.0)
    g = jnp.dot(h.astype(jnp.bfloat16), we2_ref[...],
                preferred_element_type=jnp.float32)
    x3 = jnp.dot(x.astype(jnp.bfloat16), s_ref[...],
                 preferred_element_type=jnp.float32)
    p = g * x3
    kk = p.shape[1]
    m = p[:, 0:128]
    for t in range(1, kk // 128):
        m = m + p[:, t * 128:(t + 1) * 128]
    msg = (jnp.dot(m, r4_ref[...], preferred_element_type=jnp.float32)
           + jnp.dot(x, bm_ref[...], preferred_element_type=jnp.float32))
    blk = msg.shape[0]
    o_ref[...] = jnp.concatenate(
        [msg, jnp.ones((blk, 1), jnp.float32),
         jnp.zeros((blk, 127 - DIM), jnp.float32)], axis=1)


def _gru_body(p0_ref, p1_ref, h_ref, bc_ref, wih_ref,
              bih_ref, whh_ref, bhh_ref, o_ref):
    n = o_ref.shape[0]
    d = DIM
    h = h_ref[:, 0:d]
    agg = p0_ref[0:n, 0:d] + p1_ref[0:n, 0:d]
    cnt = p0_ref[0:n, d:d + 1] + p1_ref[0:n, d:d + 1]
    deg = jnp.maximum(cnt, 1.0)
    m = jnp.maximum(agg / deg + bc_ref[...], 0.0)
    gi = jnp.dot(m, wih_ref[...], preferred_element_type=jnp.float32) + bih_ref[...]
    gh = jnp.dot(h, whh_ref[...], preferred_element_type=jnp.float32) + bhh_ref[...]
    r = jax.nn.sigmoid(gi[:, 0:d] + gh[:, 0:d])
    z = jax.nn.sigmoid(gi[:, d:2 * d] + gh[:, d:2 * d])
    nn_ = jnp.tanh(gi[:, 2 * d:3 * d] + r * gh[:, 2 * d:3 * d])
    hn = (1.0 - z) * nn_ + z * h
    o_ref[...] = jnp.concatenate(
        [hn, jnp.zeros((n, 128 - d), jnp.float32)], axis=1)


def _s2s_body(out_ref, b_ref, wih_ref, bih_ref, whh_ref, bhh_ref, w1_ref,
              b1_ref, w2_ref, b2_ref, o_ref, nb, psteps):
    d = DIM
    outv = out_ref[:, 0:d]
    bidx = b_ref[...]
    oh = (bidx == lax.broadcasted_iota(jnp.int32, (1, nb), 1)
          ).astype(jnp.float32)
    q_star = jnp.zeros((nb, 2 * d), jnp.float32)
    hl = jnp.zeros((nb, d), jnp.float32)
    cl = jnp.zeros((nb, d), jnp.float32)
    neg_inf = jnp.float32(-jnp.inf)
    for _ in range(psteps):
        gates = (jnp.dot(q_star, wih_ref[...], preferred_element_type=jnp.float32)
                 + bih_ref[...]
                 + jnp.dot(hl, whh_ref[...], preferred_element_type=jnp.float32)
                 + bhh_ref[...])
        i_g = jax.nn.sigmoid(gates[:, 0:d])
        f_g = jax.nn.sigmoid(gates[:, d:2 * d])
        g_g = jnp.tanh(gates[:, 2 * d:3 * d])
        o_g = jax.nn.sigmoid(gates[:, 3 * d:4 * d])
        cl = f_g * cl + i_g * g_g
        hl = o_g * jnp.tanh(cl)
        q = hl
        qn = jnp.dot(oh, q, preferred_element_type=jnp.float32)
        e = jnp.sum(outv * qn, axis=1, keepdims=True)
        emax = jnp.max(jnp.where(oh > 0.5, e, neg_inf), axis=0, keepdims=True)
        emax = jnp.where(emax == neg_inf, 0.0, emax)
        en = jnp.sum(oh * emax, axis=1, keepdims=True)
        a = jnp.exp(e - en)
        asum = jnp.sum(oh * a, axis=0, keepdims=True)
        an = jnp.sum(oh * asum, axis=1, keepdims=True)
        a = a / (an + 1e-16)
        r_ = lax.dot_general(oh, a * outv, (((0,), (0,)), ((), ())),
                             preferred_element_type=jnp.float32)
        q_star = jnp.concatenate([q, r_], axis=1)
    o1 = jnp.maximum(
        jnp.dot(q_star, w1_ref[...], preferred_element_type=jnp.float32)
        + b1_ref[...], 0.0)
    logits = jnp.dot(o1, w2_ref[...], preferred_element_type=jnp.float32) + b2_ref[...]
    lmax = jnp.max(logits, axis=1, keepdims=True)
    sh = logits - lmax
    lse = jnp.log(jnp.sum(jnp.exp(sh), axis=1, keepdims=True))
    o_ref[...] = sh - lse


def _run_lin0(x, w0, b0):
    n = x.shape[0]
    return pl.pallas_call(
        _lin0_body,
        out_shape=jax.ShapeDtypeStruct((n, 128), jnp.float32),
    )(x, w0, b0.reshape(1, -1))


def _run_msg(ea_p, xj, we1, be1, we2, s, r4, bm):
    ep = ea_p.shape[0]
    blk = 1024
    hid = we1.shape[1]
    kk = we2.shape[1]
    return pl.pallas_call(
        _msg_body,
        grid=(ep // blk,),
        in_specs=[
            pl.BlockSpec((blk, 2), lambda i: (i, 0)),
            pl.BlockSpec((blk, 128), lambda i: (i, 0)),
            pl.BlockSpec((2, hid), lambda i: (0, 0)),
            pl.BlockSpec((1, hid), lambda i: (0, 0)),
            pl.BlockSpec((hid, kk), lambda i: (0, 0)),
            pl.BlockSpec((DIM, kk), lambda i: (0, 0)),
            pl.BlockSpec((128, DIM), lambda i: (0, 0)),
            pl.BlockSpec((DIM, DIM), lambda i: (0, 0)),
        ],
        out_specs=pl.BlockSpec((blk, 128), lambda i: (i, 0)),
        out_shape=jax.ShapeDtypeStruct((ep, 128), jnp.float32),
    )(ea_p, xj, we1, be1.reshape(1, -1), we2.astype(jnp.bfloat16),
      s.astype(jnp.bfloat16), r4, bm)


def _run_gru(p0, p1, h, bc, wih, bih, whh, bhh):
    n = h.shape[0]
    return pl.pallas_call(
        _gru_body,
        out_shape=jax.ShapeDtypeStruct((n, 128), jnp.float32),
    )(p0, p1, h, bc.reshape(1, -1), wih, bih.reshape(1, -1),
      whh, bhh.reshape(1, -1))


def _run_s2s(out, batch2d, wih, bih, whh, bhh, w1, b1, w2, b2, psteps):
    nb = NB
    ncls = w2.shape[1]

    def body(*refs):
        _s2s_body(*refs, nb=nb, psteps=psteps)

    return pl.pallas_call(
        body,
        out_shape=jax.ShapeDtypeStruct((nb, ncls), jnp.float32),
    )(out, batch2d, wih, bih.reshape(1, -1), whh, bhh.reshape(1, -1),
      w1, b1.reshape(1, -1), w2, b2.reshape(1, -1))


def _run_gather(table, src3):
    _, cpw, _ = src3.shape
    epw = cpw * CHUNK
    ngrp = cpw // NBUF

    def body(table_ref, src_ref, out_ref, idx2, bufs, sem_g, sem_s):
        cid = lax.axis_index("c")
        sid = lax.axis_index("s")
        wid = sid * NC + cid
        base = wid * epw
        pltpu.sync_copy(src_ref.at[wid], idx2)

        def group(k, carry):
            g0 = k * NBUF
            ds = [pltpu.async_copy(table_ref.at[idx2.at[g0 + b]],
                                   bufs.at[b], sem_g)
                  for b in range(NBUF)]
            ss = []
            for b in range(NBUF):
                ds[b].wait()
                off = base + (g0 + b) * CHUNK
                ss.append(pltpu.async_copy(
                    bufs.at[b], out_ref.at[pl.ds(off, CHUNK)], sem_s))
            for s_ in ss:
                s_.wait()
            return carry

        lax.fori_loop(0, ngrp, group, 0)

    mesh = plsc.VectorSubcoreMesh(core_axis_name="c", subcore_axis_name="s")
    return pl.kernel(
        body,
        out_type=jax.ShapeDtypeStruct((NW * epw, 128), jnp.float32),
        mesh=mesh,
        scratch_types=[
            pltpu.VMEM((cpw, CHUNK), jnp.int32),
            pltpu.VMEM((NBUF, CHUNK, 128), jnp.float32),
            pltpu.SemaphoreType.DMA,
            pltpu.SemaphoreType.DMA,
        ],
    )(table, src3)


def _run_scatter(msg, dst3, z128, npad):
    ep = msg.shape[0]
    _, cpw, _ = dst3.shape
    epw = cpw * CHUNK
    nbuf = 2
    ngrp = cpw // nbuf
    rpt = npad // NS

    def body(msg_ref, dst_ref, z_ref, outa_ref, agg_sh, idx2, bufs,
             sem_m, sem_w):
        cid = lax.axis_index("c")
        sid = lax.axis_index("s")
        wid = sid * NC + cid
        r0 = sid * rpt
        pltpu.sync_copy(z_ref.at[pl.ds(r0, rpt)], agg_sh.at[pl.ds(r0, rpt)])
        pltpu.sync_copy(dst_ref.at[wid], idx2)
        plsc.subcore_barrier()
        base = wid * epw

        def group(k, carry):
            g0 = k * nbuf
            ds = [pltpu.async_copy(
                      msg_ref.at[pl.ds(base + (g0 + b) * CHUNK, CHUNK)],
                      bufs.at[b], sem_m)
                  for b in range(nbuf)]
            ws = []
            for b in range(nbuf):
                ds[b].wait()
                ws.append(pltpu.async_copy(
                    bufs.at[b], agg_sh.at[idx2.at[g0 + b]], sem_w, add=True))
            for w_ in ws:
                w_.wait()
            return carry

        lax.fori_loop(0, ngrp, group, 0)
        plsc.subcore_barrier()
        pltpu.sync_copy(agg_sh.at[pl.ds(r0, rpt)],
                        outa_ref.at[cid, pl.ds(r0, rpt)])

    mesh = plsc.VectorSubcoreMesh(core_axis_name="c", subcore_axis_name="s")
    return pl.kernel(
        body,
        out_type=jax.ShapeDtypeStruct((NC, npad, 128), jnp.float32),
        mesh=mesh,
        scratch_types=[
            pltpu.VMEM_SHARED((npad, 128), jnp.float32),
            pltpu.VMEM((cpw, CHUNK), jnp.int32),
            pltpu.VMEM((nbuf, CHUNK, 128), jnp.float32),
            pltpu.SemaphoreType.DMA,
            pltpu.SemaphoreType.DMA,
        ],
    )(msg, dst3, z128)


def kernel(x, edge_index, edge_attr, batch, W0, b0, We1, be1, We2, be2,
           b_conv, Wih_g, bih_g, Whh_g, bhh_g, Wih_l, bih_l, Whh_l, bhh_l,
           W1, b1, W2, b2):
    n = x.shape[0]
    e = edge_index.shape[1]
    kk = We2.shape[1]
    mp_steps = 2
    psteps = 4

    gran = NW * CHUNK * NBUF
    ep = -(-e // gran) * gran
    pad = ep - e
    cpw = ep // (NW * CHUNK)
    src3 = jnp.concatenate([edge_index[0], jnp.zeros((pad,), jnp.int32)]
                           ).reshape(NW, cpw, CHUNK)
    dst3 = jnp.concatenate([edge_index[1], jnp.full((pad,), n, jnp.int32)]
                           ).reshape(NW, cpw, CHUNK)
    ea_p = jnp.concatenate([edge_attr, jnp.zeros((pad, 2), jnp.float32)])
    npad = -(-(n + 1) // (NS * 8)) * (NS * 8)

    s = (jnp.arange(kk, dtype=jnp.int32)[None, :] // DIM
         == jnp.arange(DIM, dtype=jnp.int32)[:, None]).astype(jnp.float32)
    r4 = (jnp.arange(128, dtype=jnp.int32)[:, None] % DIM
          == jnp.arange(DIM, dtype=jnp.int32)[None, :]).astype(jnp.float32)
    bm = be2.reshape(DIM, DIM)
    z128 = jnp.zeros((npad, 128), jnp.float32)

    h = _run_lin0(x, W0, b0)
    for _ in range(mp_steps):
        xj = _run_gather(h, src3)
        msg = _run_msg(ea_p, xj, We1, be1, We2, s, r4, bm)
        agg2 = _run_scatter(msg, dst3, z128, npad)
        h = _run_gru(agg2[0], agg2[1], h, b_conv,
                     Wih_g, bih_g, Whh_g, bhh_g)

    return _run_s2s(h, batch.reshape(-1, 1), Wih_l, bih_l, Whh_l, bhh_l,
                    W1, b1, W2, b2, psteps)


# speedup vs baseline: 3.0902x; 1.0647x over previous
import jax
import jax.numpy as jnp
from jax import lax
from jax.experimental import pallas as pl
from jax.experimental.pallas import tpu as pltpu
from jax.experimental.pallas import tpu_sc as plsc

DIM = 32
NB = 64
NC = 2
NS = 16
NW = NC * NS
CHUNK = 128
NBUF = 4


def _lin0_body(x_ref, w_ref, b_ref, o_ref):
    n = x_ref.shape[0]
    res = jnp.maximum(
        jnp.dot(x_ref[...], w_ref[...], preferred_element_type=jnp.float32)
        + b_ref[...], 0.0)
    o_ref[...] = jnp.concatenate(
        [res, jnp.zeros((n, 128 - DIM), jnp.float32)], axis=1)


def _msg_body(ea_ref, xj_ref, we1_ref, be1_ref, we2_ref, s_ref, r4_ref,
              bm_ref, o_ref):
    ea = ea_ref[...]
    x = xj_ref[:, 0:DIM]
    h = jnp.maximum(
        ea[:, 0:1] * we1_ref[0:1, :] + ea[:, 1:2] * we1_ref[1:2, :]
        + be1_ref[...], 0.0)
    g = jnp.dot(h.astype(jnp.bfloat16), we2_ref[...],
                preferred_element_type=jnp.float32)
    x3 = jnp.dot(x.astype(jnp.bfloat16), s_ref[...],
                 preferred_element_type=jnp.float32)
    p = g * x3
    kk = p.shape[1]
    m = p[:, 0:128]
    for t in range(1, kk // 128):
        m = m + p[:, t * 128:(t + 1) * 128]
    msg = (jnp.dot(m, r4_ref[...], preferred_element_type=jnp.float32)
           + jnp.dot(x, bm_ref[...], preferred_element_type=jnp.float32))
    blk = msg.shape[0]
    o_ref[...] = jnp.concatenate(
        [msg, jnp.ones((blk, 1), jnp.float32),
         jnp.zeros((blk, 127 - DIM), jnp.float32)], axis=1)


def _gru_body(p0_ref, p1_ref, p2_ref, p3_ref, h_ref, bc_ref, wih_ref,
              bih_ref, whh_ref, bhh_ref, o_ref):
    n = o_ref.shape[0]
    d = DIM
    h = h_ref[:, 0:d]
    agg = (p0_ref[0:n, 0:d] + p1_ref[0:n, 0:d]
           + p2_ref[0:n, 0:d] + p3_ref[0:n, 0:d])
    cnt = (p0_ref[0:n, d:d + 1] + p1_ref[0:n, d:d + 1]
           + p2_ref[0:n, d:d + 1] + p3_ref[0:n, d:d + 1])
    deg = jnp.maximum(cnt, 1.0)
    m = jnp.maximum(agg / deg + bc_ref[...], 0.0)
    gi = jnp.dot(m, wih_ref[...], preferred_element_type=jnp.float32) + bih_ref[...]
    gh = jnp.dot(h, whh_ref[...], preferred_element_type=jnp.float32) + bhh_ref[...]
    r = jax.nn.sigmoid(gi[:, 0:d] + gh[:, 0:d])
    z = jax.nn.sigmoid(gi[:, d:2 * d] + gh[:, d:2 * d])
    nn_ = jnp.tanh(gi[:, 2 * d:3 * d] + r * gh[:, 2 * d:3 * d])
    hn = (1.0 - z) * nn_ + z * h
    o_ref[...] = jnp.concatenate(
        [hn, jnp.zeros((n, 128 - d), jnp.float32)], axis=1)


def _s2s_body(out_ref, b_ref, wih_ref, bih_ref, whh_ref, bhh_ref, w1_ref,
              b1_ref, w2_ref, b2_ref, o_ref, nb, psteps):
    d = DIM
    outv = out_ref[:, 0:d]
    bidx = b_ref[...]
    oh = (bidx == lax.broadcasted_iota(jnp.int32, (1, nb), 1)
          ).astype(jnp.float32)
    q_star = jnp.zeros((nb, 2 * d), jnp.float32)
    hl = jnp.zeros((nb, d), jnp.float32)
    cl = jnp.zeros((nb, d), jnp.float32)
    neg_inf = jnp.float32(-jnp.inf)
    for _ in range(psteps):
        gates = (jnp.dot(q_star, wih_ref[...], preferred_element_type=jnp.float32)
                 + bih_ref[...]
                 + jnp.dot(hl, whh_ref[...], preferred_element_type=jnp.float32)
                 + bhh_ref[...])
        i_g = jax.nn.sigmoid(gates[:, 0:d])
        f_g = jax.nn.sigmoid(gates[:, d:2 * d])
        g_g = jnp.tanh(gates[:, 2 * d:3 * d])
        o_g = jax.nn.sigmoid(gates[:, 3 * d:4 * d])
        cl = f_g * cl + i_g * g_g
        hl = o_g * jnp.tanh(cl)
        q = hl
        qn = jnp.dot(oh, q, preferred_element_type=jnp.float32)
        e = jnp.sum(outv * qn, axis=1, keepdims=True)
        emax = jnp.max(jnp.where(oh > 0.5, e, neg_inf), axis=0, keepdims=True)
        emax = jnp.where(emax == neg_inf, 0.0, emax)
        en = jnp.sum(oh * emax, axis=1, keepdims=True)
        a = jnp.exp(e - en)
        asum = jnp.sum(oh * a, axis=0, keepdims=True)
        an = jnp.sum(oh * asum, axis=1, keepdims=True)
        a = a / (an + 1e-16)
        r_ = lax.dot_general(oh, a * outv, (((0,), (0,)), ((), ())),
                             preferred_element_type=jnp.float32)
        q_star = jnp.concatenate([q, r_], axis=1)
    o1 = jnp.maximum(
        jnp.dot(q_star, w1_ref[...], preferred_element_type=jnp.float32)
        + b1_ref[...], 0.0)
    logits = jnp.dot(o1, w2_ref[...], preferred_element_type=jnp.float32) + b2_ref[...]
    lmax = jnp.max(logits, axis=1, keepdims=True)
    sh = logits - lmax
    lse = jnp.log(jnp.sum(jnp.exp(sh), axis=1, keepdims=True))
    o_ref[...] = sh - lse


def _run_lin0(x, w0, b0):
    n = x.shape[0]
    return pl.pallas_call(
        _lin0_body,
        out_shape=jax.ShapeDtypeStruct((n, 128), jnp.float32),
    )(x, w0, b0.reshape(1, -1))


def _run_msg(ea_p, xj, we1, be1, we2, s, r4, bm):
    ep = ea_p.shape[0]
    blk = 1024
    hid = we1.shape[1]
    kk = we2.shape[1]
    return pl.pallas_call(
        _msg_body,
        grid=(ep // blk,),
        in_specs=[
            pl.BlockSpec((blk, 2), lambda i: (i, 0)),
            pl.BlockSpec((blk, 128), lambda i: (i, 0)),
            pl.BlockSpec((2, hid), lambda i: (0, 0)),
            pl.BlockSpec((1, hid), lambda i: (0, 0)),
            pl.BlockSpec((hid, kk), lambda i: (0, 0)),
            pl.BlockSpec((DIM, kk), lambda i: (0, 0)),
            pl.BlockSpec((128, DIM), lambda i: (0, 0)),
            pl.BlockSpec((DIM, DIM), lambda i: (0, 0)),
        ],
        out_specs=pl.BlockSpec((blk, 128), lambda i: (i, 0)),
        out_shape=jax.ShapeDtypeStruct((ep, 128), jnp.float32),
    )(ea_p, xj, we1, be1.reshape(1, -1), we2.astype(jnp.bfloat16),
      s.astype(jnp.bfloat16), r4, bm)


def _run_gru(p0, p1, p2, p3, h, bc, wih, bih, whh, bhh):
    n = h.shape[0]
    return pl.pallas_call(
        _gru_body,
        out_shape=jax.ShapeDtypeStruct((n, 128), jnp.float32),
    )(p0, p1, p2, p3, h, bc.reshape(1, -1), wih, bih.reshape(1, -1),
      whh, bhh.reshape(1, -1))


def _run_s2s(out, batch2d, wih, bih, whh, bhh, w1, b1, w2, b2, psteps):
    nb = NB
    ncls = w2.shape[1]

    def body(*refs):
        _s2s_body(*refs, nb=nb, psteps=psteps)

    return pl.pallas_call(
        body,
        out_shape=jax.ShapeDtypeStruct((nb, ncls), jnp.float32),
    )(out, batch2d, wih, bih.reshape(1, -1), whh, bhh.reshape(1, -1),
      w1, b1.reshape(1, -1), w2, b2.reshape(1, -1))


def _run_gather(table, src3):
    _, cpw, _ = src3.shape
    epw = cpw * CHUNK
    ngrp = cpw // NBUF

    def body(table_ref, src_ref, out_ref, idx2, bufs, sem_g, sem_s):
        cid = lax.axis_index("c")
        sid = lax.axis_index("s")
        wid = sid * NC + cid
        base = wid * epw
        pltpu.sync_copy(src_ref.at[wid], idx2)

        def group(k, carry):
            g0 = k * NBUF
            ds = [pltpu.async_copy(table_ref.at[idx2.at[g0 + b]],
                                   bufs.at[b], sem_g)
                  for b in range(NBUF)]
            ss = []
            for b in range(NBUF):
                ds[b].wait()
                off = base + (g0 + b) * CHUNK
                ss.append(pltpu.async_copy(
                    bufs.at[b], out_ref.at[pl.ds(off, CHUNK)], sem_s))
            for s_ in ss:
                s_.wait()
            return carry

        lax.fori_loop(0, ngrp, group, 0)

    mesh = plsc.VectorSubcoreMesh(core_axis_name="c", subcore_axis_name="s")
    return pl.kernel(
        body,
        out_type=jax.ShapeDtypeStruct((NW * epw, 128), jnp.float32),
        mesh=mesh,
        scratch_types=[
            pltpu.VMEM((cpw, CHUNK), jnp.int32),
            pltpu.VMEM((NBUF, CHUNK, 128), jnp.float32),
            pltpu.SemaphoreType.DMA,
            pltpu.SemaphoreType.DMA,
        ],
    )(table, src3)


def _run_scatter(msg, dst3, z128, npad):
    ep = msg.shape[0]
    _, cpw, _ = dst3.shape
    epw = cpw * CHUNK
    nbuf = 2
    ngrp = cpw // nbuf
    rpt = npad // NS

    def body(msg_ref, dst_ref, z_ref, outa_ref, agg_sh, idx2, bufs,
             sem_m, sem_w):
        cid = lax.axis_index("c")
        sid = lax.axis_index("s")
        wid = sid * NC + cid
        r0 = sid * rpt
        pltpu.sync_copy(z_ref.at[pl.ds(r0, rpt)], agg_sh.at[pl.ds(r0, rpt)])
        pltpu.sync_copy(dst_ref.at[wid], idx2)
        plsc.subcore_barrier()
        base = wid * epw

        def group(k, carry):
            g0 = k * nbuf
            ds = [pltpu.async_copy(
                      msg_ref.at[pl.ds(base + (g0 + b) * CHUNK, CHUNK)],
                      bufs.at[b], sem_m)
                  for b in range(nbuf)]
            ws = []
            for b in range(nbuf):
                ds[b].wait()
                ws.append(pltpu.async_copy(
                    bufs.at[b], agg_sh.at[idx2.at[g0 + b]], sem_w, add=True))
            for w_ in ws:
                w_.wait()
            return carry

        lax.fori_loop(0, ngrp, group, 0)
        plsc.subcore_barrier()
        pltpu.sync_copy(agg_sh.at[pl.ds(r0, rpt)],
                        outa_ref.at[cid, pl.ds(r0, rpt)])

    mesh = plsc.VectorSubcoreMesh(core_axis_name="c", subcore_axis_name="s")
    return pl.kernel(
        body,
        out_type=jax.ShapeDtypeStruct((NC, npad, 128), jnp.float32),
        mesh=mesh,
        scratch_types=[
            pltpu.VMEM_SHARED((npad, 128), jnp.float32),
            pltpu.VMEM((cpw, CHUNK), jnp.int32),
            pltpu.VMEM((nbuf, CHUNK, 128), jnp.float32),
            pltpu.SemaphoreType.DMA,
            pltpu.SemaphoreType.DMA,
        ],
    )(msg, dst3, z128)


def kernel(x, edge_index, edge_attr, batch, W0, b0, We1, be1, We2, be2,
           b_conv, Wih_g, bih_g, Whh_g, bhh_g, Wih_l, bih_l, Whh_l, bhh_l,
           W1, b1, W2, b2):
    n = x.shape[0]
    e = edge_index.shape[1]
    kk = We2.shape[1]
    mp_steps = 2
    psteps = 4

    gran = 2 * NW * CHUNK * NBUF
    ep = -(-e // gran) * gran
    pad = ep - e
    half = ep // 2
    cpw = half // (NW * CHUNK)
    src_p = jnp.concatenate([edge_index[0], jnp.zeros((pad,), jnp.int32)])
    dst_p = jnp.concatenate([edge_index[1], jnp.full((pad,), n, jnp.int32)])
    src3 = [src_p[:half].reshape(NW, cpw, CHUNK),
            src_p[half:].reshape(NW, cpw, CHUNK)]
    dst3 = [dst_p[:half].reshape(NW, cpw, CHUNK),
            dst_p[half:].reshape(NW, cpw, CHUNK)]
    ea_p = jnp.concatenate([edge_attr, jnp.zeros((pad, 2), jnp.float32)])
    ea_s = [ea_p[:half], ea_p[half:]]
    npad = -(-(n + 1) // (NS * 8)) * (NS * 8)

    s = (jnp.arange(kk, dtype=jnp.int32)[None, :] // DIM
         == jnp.arange(DIM, dtype=jnp.int32)[:, None]).astype(jnp.float32)
    r4 = (jnp.arange(128, dtype=jnp.int32)[:, None] % DIM
          == jnp.arange(DIM, dtype=jnp.int32)[None, :]).astype(jnp.float32)
    bm = be2.reshape(DIM, DIM)
    z128 = jnp.zeros((npad, 128), jnp.float32)

    h = _run_lin0(x, W0, b0)
    for _ in range(mp_steps):
        xj_a = _run_gather(h, src3[0])
        xj_b = _run_gather(h, src3[1])
        msg_a = _run_msg(ea_s[0], xj_a, We1, be1, We2, s, r4, bm)
        msg_b = _run_msg(ea_s[1], xj_b, We1, be1, We2, s, r4, bm)
        agg_a = _run_scatter(msg_a, dst3[0], z128, npad)
        agg_b = _run_scatter(msg_b, dst3[1], z128, npad)
        h = _run_gru(agg_a[0], agg_a[1], agg_b[0], agg_b[1], h, b_conv,
                     Wih_g, bih_g, Whh_g, bhh_g)

    return _run_s2s(h, batch.reshape(-1, 1), Wih_l, bih_l, Whh_l, bhh_l,
                    W1, b1, W2, b2, psteps)


# speedup vs baseline: 4.1084x; 1.3295x over previous
import jax
import jax.numpy as jnp
from jax import lax
from jax.experimental import pallas as pl
from jax.experimental.pallas import tpu as pltpu
from jax.experimental.pallas import tpu_sc as plsc

DIM = 32
NB = 64
NC = 2
NS = 16
NW = NC * NS
CHUNK = 128
NBUF = 4


def _lin0_body(x_ref, w_ref, b_ref, o_ref):
    n = x_ref.shape[0]
    res = jnp.maximum(
        jnp.dot(x_ref[...], w_ref[...], preferred_element_type=jnp.float32)
        + b_ref[...], 0.0)
    o_ref[...] = jnp.concatenate(
        [res, jnp.zeros((n, 128 - DIM), jnp.float32)], axis=1)


def _msg_body(ea_ref, xj_ref, we1_ref, be1_ref, we2_ref, s_ref, r4_ref,
              bm_ref, o_ref):
    ea = ea_ref[...]
    x = xj_ref[:, 0:DIM]
    h = jnp.maximum(
        ea[:, 0:1] * we1_ref[0:1, :] + ea[:, 1:2] * we1_ref[1:2, :]
        + be1_ref[...], 0.0)
    g = jnp.dot(h.astype(jnp.bfloat16), we2_ref[...],
                preferred_element_type=jnp.float32)
    x3 = jnp.dot(x.astype(jnp.bfloat16), s_ref[...],
                 preferred_element_type=jnp.float32)
    p = g * x3
    kk = p.shape[1]
    m = p[:, 0:128]
    for t in range(1, kk // 128):
        m = m + p[:, t * 128:(t + 1) * 128]
    msg = (jnp.dot(m, r4_ref[...], preferred_element_type=jnp.float32)
           + jnp.dot(x, bm_ref[...], preferred_element_type=jnp.float32))
    blk = msg.shape[0]
    o_ref[...] = jnp.concatenate(
        [msg, jnp.ones((blk, 1), jnp.float32),
         jnp.zeros((blk, 127 - DIM), jnp.float32)], axis=1)


def _gru_body(p0_ref, p1_ref, p2_ref, p3_ref, h_ref, bc_ref, wih_ref,
              bih_ref, whh_ref, bhh_ref, o_ref):
    n = o_ref.shape[0]
    d = DIM
    h = h_ref[:, 0:d]
    agg = (p0_ref[0:n, 0:d] + p1_ref[0:n, 0:d]
           + p2_ref[0:n, 0:d] + p3_ref[0:n, 0:d])
    cnt = (p0_ref[0:n, d:d + 1] + p1_ref[0:n, d:d + 1]
           + p2_ref[0:n, d:d + 1] + p3_ref[0:n, d:d + 1])
    deg = jnp.maximum(cnt, 1.0)
    m = jnp.maximum(agg / deg + bc_ref[...], 0.0)
    gi = jnp.dot(m, wih_ref[...], preferred_element_type=jnp.float32) + bih_ref[...]
    gh = jnp.dot(h, whh_ref[...], preferred_element_type=jnp.float32) + bhh_ref[...]
    r = jax.nn.sigmoid(gi[:, 0:d] + gh[:, 0:d])
    z = jax.nn.sigmoid(gi[:, d:2 * d] + gh[:, d:2 * d])
    nn_ = jnp.tanh(gi[:, 2 * d:3 * d] + r * gh[:, 2 * d:3 * d])
    hn = (1.0 - z) * nn_ + z * h
    o_ref[...] = jnp.concatenate(
        [hn, jnp.zeros((n, 128 - d), jnp.float32)], axis=1)


def _s2s_body(out_ref, b_ref, wih_ref, bih_ref, whh_ref, bhh_ref, w1_ref,
              b1_ref, w2_ref, b2_ref, o_ref, nb, psteps):
    d = DIM
    outv = out_ref[:, 0:d]
    bidx = b_ref[...]
    oh = (bidx == lax.broadcasted_iota(jnp.int32, (1, nb), 1)
          ).astype(jnp.float32)
    q_star = jnp.zeros((nb, 2 * d), jnp.float32)
    hl = jnp.zeros((nb, d), jnp.float32)
    cl = jnp.zeros((nb, d), jnp.float32)
    neg_inf = jnp.float32(-jnp.inf)
    for _ in range(psteps):
        gates = (jnp.dot(q_star, wih_ref[...], preferred_element_type=jnp.float32)
                 + bih_ref[...]
                 + jnp.dot(hl, whh_ref[...], preferred_element_type=jnp.float32)
                 + bhh_ref[...])
        i_g = jax.nn.sigmoid(gates[:, 0:d])
        f_g = jax.nn.sigmoid(gates[:, d:2 * d])
        g_g = jnp.tanh(gates[:, 2 * d:3 * d])
        o_g = jax.nn.sigmoid(gates[:, 3 * d:4 * d])
        cl = f_g * cl + i_g * g_g
        hl = o_g * jnp.tanh(cl)
        q = hl
        qn = jnp.dot(oh, q, preferred_element_type=jnp.float32)
        e = jnp.sum(outv * qn, axis=1, keepdims=True)
        emax = jnp.max(jnp.where(oh > 0.5, e, neg_inf), axis=0, keepdims=True)
        emax = jnp.where(emax == neg_inf, 0.0, emax)
        en = jnp.sum(oh * emax, axis=1, keepdims=True)
        a = jnp.exp(e - en)
        asum = jnp.sum(oh * a, axis=0, keepdims=True)
        an = jnp.sum(oh * asum, axis=1, keepdims=True)
        a = a / (an + 1e-16)
        r_ = lax.dot_general(oh, a * outv, (((0,), (0,)), ((), ())),
                             preferred_element_type=jnp.float32)
        q_star = jnp.concatenate([q, r_], axis=1)
    o1 = jnp.maximum(
        jnp.dot(q_star, w1_ref[...], preferred_element_type=jnp.float32)
        + b1_ref[...], 0.0)
    logits = jnp.dot(o1, w2_ref[...], preferred_element_type=jnp.float32) + b2_ref[...]
    lmax = jnp.max(logits, axis=1, keepdims=True)
    sh = logits - lmax
    lse = jnp.log(jnp.sum(jnp.exp(sh), axis=1, keepdims=True))
    o_ref[...] = sh - lse


def _run_lin0(x, w0, b0):
    n = x.shape[0]
    return pl.pallas_call(
        _lin0_body,
        out_shape=jax.ShapeDtypeStruct((n, 128), jnp.float32),
    )(x, w0, b0.reshape(1, -1))


def _run_msg(ea_p, xj, we1, be1, we2, s, r4, bm):
    ep = ea_p.shape[0]
    blk = 1024
    hid = we1.shape[1]
    kk = we2.shape[1]
    return pl.pallas_call(
        _msg_body,
        grid=(ep // blk,),
        in_specs=[
            pl.BlockSpec((blk, 2), lambda i: (i, 0)),
            pl.BlockSpec((blk, 128), lambda i: (i, 0)),
            pl.BlockSpec((2, hid), lambda i: (0, 0)),
            pl.BlockSpec((1, hid), lambda i: (0, 0)),
            pl.BlockSpec((hid, kk), lambda i: (0, 0)),
            pl.BlockSpec((DIM, kk), lambda i: (0, 0)),
            pl.BlockSpec((128, DIM), lambda i: (0, 0)),
            pl.BlockSpec((DIM, DIM), lambda i: (0, 0)),
        ],
        out_specs=pl.BlockSpec((blk, 128), lambda i: (i, 0)),
        out_shape=jax.ShapeDtypeStruct((ep, 128), jnp.float32),
    )(ea_p, xj, we1, be1.reshape(1, -1), we2.astype(jnp.bfloat16),
      s.astype(jnp.bfloat16), r4, bm)


def _run_gru(p0, p1, p2, p3, h, bc, wih, bih, whh, bhh):
    n = h.shape[0]
    return pl.pallas_call(
        _gru_body,
        out_shape=jax.ShapeDtypeStruct((n, 128), jnp.float32),
    )(p0, p1, p2, p3, h, bc.reshape(1, -1), wih, bih.reshape(1, -1),
      whh, bhh.reshape(1, -1))


def _run_s2s(out, batch2d, wih, bih, whh, bhh, w1, b1, w2, b2, psteps):
    nb = NB
    ncls = w2.shape[1]

    def body(*refs):
        _s2s_body(*refs, nb=nb, psteps=psteps)

    return pl.pallas_call(
        body,
        out_shape=jax.ShapeDtypeStruct((nb, ncls), jnp.float32),
    )(out, batch2d, wih, bih.reshape(1, -1), whh, bhh.reshape(1, -1),
      w1, b1.reshape(1, -1), w2, b2.reshape(1, -1))


def _run_gather(table, src3):
    n = table.shape[0]
    _, cpw, _ = src3.shape
    epw = cpw * CHUNK
    nbuf = 2
    ngrp = cpw // nbuf
    rp = -(-n // (NS * 8)) * 8
    last = n - rp * (NS - 1)

    def body(table_ref, src_ref, out_ref, tb_sh, idx2, bufs, sem_g, sem_s):
        cid = lax.axis_index("c")
        sid = lax.axis_index("s")
        wid = sid * NC + cid
        base = wid * epw

        @pl.when(sid < NS - 1)
        def _():
            pltpu.sync_copy(table_ref.at[pl.ds(sid * rp, rp)],
                            tb_sh.at[pl.ds(sid * rp, rp)])

        @pl.when(sid == NS - 1)
        def _():
            pltpu.sync_copy(table_ref.at[pl.ds((NS - 1) * rp, last)],
                            tb_sh.at[pl.ds((NS - 1) * rp, last)])

        pltpu.sync_copy(src_ref.at[wid], idx2)
        plsc.subcore_barrier()

        def group(k, carry):
            g0 = k * nbuf
            ds = [pltpu.async_copy(tb_sh.at[idx2.at[g0 + b]],
                                   bufs.at[b], sem_g)
                  for b in range(nbuf)]
            ss = []
            for b in range(nbuf):
                ds[b].wait()
                off = base + (g0 + b) * CHUNK
                ss.append(pltpu.async_copy(
                    bufs.at[b], out_ref.at[pl.ds(off, CHUNK)], sem_s))
            for s_ in ss:
                s_.wait()
            return carry

        lax.fori_loop(0, ngrp, group, 0)

    mesh = plsc.VectorSubcoreMesh(core_axis_name="c", subcore_axis_name="s")
    return pl.kernel(
        body,
        out_type=jax.ShapeDtypeStruct((NW * epw, 128), jnp.float32),
        mesh=mesh,
        scratch_types=[
            pltpu.VMEM_SHARED((n, 128), jnp.float32),
            pltpu.VMEM((cpw, CHUNK), jnp.int32),
            pltpu.VMEM((nbuf, CHUNK, 128), jnp.float32),
            pltpu.SemaphoreType.DMA,
            pltpu.SemaphoreType.DMA,
        ],
    )(table, src3)


def _run_scatter(msg, dst3, z128, npad):
    ep = msg.shape[0]
    _, cpw, _ = dst3.shape
    epw = cpw * CHUNK
    nbuf = 2
    ngrp = cpw // nbuf
    rpt = npad // NS

    def body(msg_ref, dst_ref, z_ref, outa_ref, agg_sh, idx2, bufs,
             sem_m, sem_w):
        cid = lax.axis_index("c")
        sid = lax.axis_index("s")
        wid = sid * NC + cid
        r0 = sid * rpt
        pltpu.sync_copy(z_ref.at[pl.ds(r0, rpt)], agg_sh.at[pl.ds(r0, rpt)])
        pltpu.sync_copy(dst_ref.at[wid], idx2)
        plsc.subcore_barrier()
        base = wid * epw

        def group(k, carry):
            g0 = k * nbuf
            ds = [pltpu.async_copy(
                      msg_ref.at[pl.ds(base + (g0 + b) * CHUNK, CHUNK)],
                      bufs.at[b], sem_m)
                  for b in range(nbuf)]
            ws = []
            for b in range(nbuf):
                ds[b].wait()
                ws.append(pltpu.async_copy(
                    bufs.at[b], agg_sh.at[idx2.at[g0 + b]], sem_w, add=True))
            for w_ in ws:
                w_.wait()
            return carry

        lax.fori_loop(0, ngrp, group, 0)
        plsc.subcore_barrier()
        pltpu.sync_copy(agg_sh.at[pl.ds(r0, rpt)],
                        outa_ref.at[cid, pl.ds(r0, rpt)])

    mesh = plsc.VectorSubcoreMesh(core_axis_name="c", subcore_axis_name="s")
    return pl.kernel(
        body,
        out_type=jax.ShapeDtypeStruct((NC, npad, 128), jnp.float32),
        mesh=mesh,
        scratch_types=[
            pltpu.VMEM_SHARED((npad, 128), jnp.float32),
            pltpu.VMEM((cpw, CHUNK), jnp.int32),
            pltpu.VMEM((nbuf, CHUNK, 128), jnp.float32),
            pltpu.SemaphoreType.DMA,
            pltpu.SemaphoreType.DMA,
        ],
    )(msg, dst3, z128)


def kernel(x, edge_index, edge_attr, batch, W0, b0, We1, be1, We2, be2,
           b_conv, Wih_g, bih_g, Whh_g, bhh_g, Wih_l, bih_l, Whh_l, bhh_l,
           W1, b1, W2, b2):
    n = x.shape[0]
    e = edge_index.shape[1]
    kk = We2.shape[1]
    mp_steps = 2
    psteps = 4

    gran = 2 * NW * CHUNK * NBUF
    ep = -(-e // gran) * gran
    pad = ep - e
    half = ep // 2
    cpw = half // (NW * CHUNK)
    src_p = jnp.concatenate([edge_index[0], jnp.zeros((pad,), jnp.int32)])
    dst_p = jnp.concatenate([edge_index[1], jnp.full((pad,), n, jnp.int32)])
    src3 = [src_p[:half].reshape(NW, cpw, CHUNK),
            src_p[half:].reshape(NW, cpw, CHUNK)]
    dst3 = [dst_p[:half].reshape(NW, cpw, CHUNK),
            dst_p[half:].reshape(NW, cpw, CHUNK)]
    ea_p = jnp.concatenate([edge_attr, jnp.zeros((pad, 2), jnp.float32)])
    ea_s = [ea_p[:half], ea_p[half:]]
    npad = -(-(n + 1) // (NS * 8)) * (NS * 8)

    s = (jnp.arange(kk, dtype=jnp.int32)[None, :] // DIM
         == jnp.arange(DIM, dtype=jnp.int32)[:, None]).astype(jnp.float32)
    r4 = (jnp.arange(128, dtype=jnp.int32)[:, None] % DIM
          == jnp.arange(DIM, dtype=jnp.int32)[None, :]).astype(jnp.float32)
    bm = be2.reshape(DIM, DIM)
    z128 = jnp.zeros((npad, 128), jnp.float32)

    h = _run_lin0(x, W0, b0)
    for _ in range(mp_steps):
        xj_a = _run_gather(h, src3[0])
        xj_b = _run_gather(h, src3[1])
        msg_a = _run_msg(ea_s[0], xj_a, We1, be1, We2, s, r4, bm)
        msg_b = _run_msg(ea_s[1], xj_b, We1, be1, We2, s, r4, bm)
        agg_a = _run_scatter(msg_a, dst3[0], z128, npad)
        agg_b = _run_scatter(msg_b, dst3[1], z128, npad)
        h = _run_gru(agg_a[0], agg_a[1], agg_b[0], agg_b[1], h, b_conv,
                     Wih_g, bih_g, Whh_g, bhh_g)

    return _run_s2s(h, batch.reshape(-1, 1), Wih_l, bih_l, Whh_l, bhh_l,
                    W1, b1, W2, b2, psteps)
